# baseline (speedup 1.0000x reference)
import functools

import jax
import jax.numpy as jnp
from jax import lax
from jax.experimental import pallas as pl
from jax.experimental.pallas import tpu as pltpu
from jax.experimental.pallas import tpu_sc as plsc

_E = 16
_K = 2
_NC = 2
_NS = 16
_NW = _NC * _NS
_RSC = 128
_NB = 8


def _gate_rows(p, iot):
    m1 = jnp.max(p, axis=1, keepdims=True)
    i1 = jnp.min(jnp.where(p == m1, iot, _E), axis=1, keepdims=True)
    h1 = iot == i1
    p2 = jnp.where(h1, -1.0, p)
    m2 = jnp.max(p2, axis=1, keepdims=True)
    i2 = jnp.min(jnp.where(p2 == m2, iot, _E), axis=1, keepdims=True)
    w = jnp.where(h1 | (iot == i2), p, 0.0)
    return w, i1, i2


def _tc_body(x_ref, w_ref, o_ref, i_ref):
    l = lax.dot_general(
        x_ref[...], w_ref[...],
        (((1,), (1,)), ((), ())),
        preferred_element_type=jnp.float32,
    )
    iot = lax.broadcasted_iota(jnp.int32, l.shape, 1)
    m = jnp.max(l, axis=1, keepdims=True)
    ex = jnp.exp(l - m)
    p = ex / jnp.sum(ex, axis=1, keepdims=True)
    w, i1, i2 = _gate_rows(p, iot)
    o_ref[...] = w
    i_ref[...] = jnp.concatenate([i1, i2], axis=1)


def _tc_gate(x, W, rows):
    N = x.shape[1]
    bm = rows // _NB
    return pl.pallas_call(
        _tc_body,
        grid=(_NB,),
        in_specs=[
            pl.BlockSpec((bm, N), lambda i: (i, 0)),
            pl.BlockSpec((_E, N), lambda i: (0, 0)),
        ],
        out_specs=[
            pl.BlockSpec((bm, _E), lambda i: (i, 0)),
            pl.BlockSpec((bm, _K), lambda i: (i, 0)),
        ],
        out_shape=[
            jax.ShapeDtypeStruct((rows, _E), jnp.float32),
            jax.ShapeDtypeStruct((rows, _K), jnp.int32),
        ],
    )(x, W)


def _sc_body(row0, rpw, x_hbm, wt_hbm, w_hbm, i_hbm, x_v, wt_v, w_v, i_v):
    wid = lax.axis_index("s") * _NC + lax.axis_index("c")
    base = row0 + wid * rpw
    pltpu.sync_copy(x_hbm.at[pl.ds(base, rpw)], x_v)
    pltpu.sync_copy(wt_hbm, wt_v)
    iota = lax.iota(jnp.int32, 16)
    n = x_v.shape[1]
    nseg = n // 256

    def row(r, carry):
        accs = [jnp.zeros((16,), jnp.float32) for _ in range(_E)]

        def seg(sg, accs):
            k0 = sg * 256
            xs = [x_v[r, pl.ds(k0 + 16 * j, 16)] for j in range(16)]
            out = []
            for e in range(_E):
                a = accs[e]
                for j in range(16):
                    a = a + xs[j] * wt_v[e, pl.ds(k0 + 16 * j, 16)]
                out.append(a)
            return out

        accs = lax.fori_loop(0, nseg, seg, accs)
        l = jnp.zeros((16,), jnp.float32)
        for e in range(_E):
            l = jnp.where(iota == e, jnp.sum(accs[e], axis=0), l)
        m = jnp.max(l, axis=0)
        ex = jnp.exp(l - m)
        p = ex / jnp.sum(ex, axis=0)
        m1 = jnp.max(p, axis=0)
        i1 = jnp.min(jnp.where(p == m1, iota, _E), axis=0)
        h1 = iota == i1
        p2 = jnp.where(h1, -1.0, p)
        m2 = jnp.max(p2, axis=0)
        i2 = jnp.min(jnp.where(p2 == m2, iota, _E), axis=0)
        w_v[r, :] = jnp.where(h1 | (iota == i2), p, 0.0)
        pair = jnp.where(iota == 0, i1, i2)
        plsc.store_scatter(i_v, [2 * r + iota], pair, mask=iota < _K)
        return carry

    lax.fori_loop(0, rpw, row, 0)
    pltpu.sync_copy(w_v, w_hbm.at[pl.ds(wid * rpw, rpw)])
    pltpu.sync_copy(i_v, i_hbm.at[pl.ds(wid * rpw * _K, rpw * _K)])


def _sc_gate(x, W, row0):
    N = x.shape[1]
    rpw = _RSC // _NW
    mesh = plsc.VectorSubcoreMesh(core_axis_name="c", subcore_axis_name="s")
    weights, idx_flat = pl.kernel(
        functools.partial(_sc_body, row0, rpw),
        out_type=[
            jax.ShapeDtypeStruct((_RSC, _E), jnp.float32),
            jax.ShapeDtypeStruct((_RSC * _K,), jnp.int32),
        ],
        mesh=mesh,
        compiler_params=pltpu.CompilerParams(
            needs_layout_passes=False, skip_device_barrier=True
        ),
        scratch_types=[
            pltpu.VMEM((rpw, N), jnp.float32),
            pltpu.VMEM((_E, N), jnp.float32),
            pltpu.VMEM((rpw, _E), jnp.float32),
            pltpu.VMEM((rpw * _K,), jnp.int32),
        ],
    )(x, W)
    return weights, idx_flat.reshape(_RSC, _K)


def kernel(x, W):
    T = x.shape[0]
    rows_tc = T - _RSC
    w_sc, i_sc = _sc_gate(x, W, rows_tc)
    w_tc, i_tc = _tc_gate(x, W, rows_tc)
    weights = jnp.concatenate([w_tc, w_sc], axis=0)
    idx = jnp.concatenate([i_tc, i_sc], axis=0)
    return weights, idx


# speedup vs baseline: 1.0037x; 1.0037x over previous
import functools

import jax
import jax.numpy as jnp
from jax import lax
from jax.experimental import pallas as pl
from jax.experimental.pallas import tpu as pltpu
from jax.experimental.pallas import tpu_sc as plsc

_E = 16
_K = 2
_NC = 2
_NS = 16
_NW = _NC * _NS
_BM = 1024


def _logits_body(x_ref, w_ref, o_ref):
    o_ref[...] = lax.dot_general(
        x_ref[...], w_ref[...],
        (((1,), (1,)), ((), ())),
        preferred_element_type=jnp.float32,
    )


def _logits(x, W):
    T, N = x.shape
    return pl.pallas_call(
        _logits_body,
        grid=(T // _BM,),
        in_specs=[
            pl.BlockSpec((_BM, N), lambda i: (i, 0)),
            pl.BlockSpec((_E, N), lambda i: (0, 0)),
        ],
        out_specs=pl.BlockSpec((_BM, _E), lambda i: (i, 0)),
        out_shape=jax.ShapeDtypeStruct((T, _E), jnp.float32),
    )(x, W)


def _route_body(rw, logits_hbm, w_hbm, i_hbm, lg_v, w_v, i_v):
    wid = lax.axis_index("s") * _NC + lax.axis_index("c")
    base = wid * rw
    pltpu.sync_copy(logits_hbm.at[pl.ds(base, rw)], lg_v)
    iota = lax.iota(jnp.int32, 16)
    zeros = jnp.zeros((16,), jnp.float32)

    for t in range(rw // 16):
        row = t * 16 + iota
        ls = [
            plsc.load_gather(lg_v, [row, jnp.full((16,), e, jnp.int32)])
            for e in range(_E)
        ]
        m = ls[0]
        for e in range(1, _E):
            m = jnp.maximum(m, ls[e])
        es = [jnp.exp(l - m) for l in ls]
        s = es[0]
        for e in range(1, _E):
            s = s + es[e]
        inv = 1.0 / s
        m1 = es[0] * inv
        i1 = jnp.zeros((16,), jnp.int32)
        m2 = jnp.full((16,), -1.0, jnp.float32)
        i2 = jnp.zeros((16,), jnp.int32)
        for e in range(1, _E):
            p = es[e] * inv
            gt1 = p > m1
            gt2 = p > m2
            i2 = jnp.where(gt1, i1, jnp.where(gt2, e, i2))
            m2 = jnp.where(gt1, m1, jnp.where(gt2, p, m2))
            i1 = jnp.where(gt1, e, i1)
            m1 = jnp.where(gt1, p, m1)
        for j in range(16):
            w_v[t * 16 + j, :] = zeros
        plsc.store_scatter(w_v, [row, i1], m1)
        plsc.store_scatter(w_v, [row, i2], m2)
        plsc.store_scatter(i_v, [row * _K], i1)
        plsc.store_scatter(i_v, [row * _K + 1], i2)

    pltpu.sync_copy(w_v, w_hbm.at[pl.ds(base, rw)])
    pltpu.sync_copy(i_v, i_hbm.at[pl.ds(base * _K, rw * _K)])


def _route(logits):
    T = logits.shape[0]
    rw = T // _NW
    mesh = plsc.VectorSubcoreMesh(core_axis_name="c", subcore_axis_name="s")
    weights, idx_flat = pl.kernel(
        functools.partial(_route_body, rw),
        out_type=[
            jax.ShapeDtypeStruct((T, _E), jnp.float32),
            jax.ShapeDtypeStruct((T * _K,), jnp.int32),
        ],
        mesh=mesh,
        compiler_params=pltpu.CompilerParams(
            needs_layout_passes=False, skip_device_barrier=True
        ),
        scratch_types=[
            pltpu.VMEM((rw, _E), jnp.float32),
            pltpu.VMEM((rw, _E), jnp.float32),
            pltpu.VMEM((rw * _K,), jnp.int32),
        ],
    )(logits)
    return weights, idx_flat.reshape(T, _K)


def kernel(x, W):
    logits = _logits(x, W)
    return _route(logits)


# speedup vs baseline: 1.0433x; 1.0395x over previous
import functools

import jax
import jax.numpy as jnp
from jax import lax
from jax.experimental import pallas as pl
from jax.experimental.pallas import tpu as pltpu
from jax.experimental.pallas import tpu_sc as plsc

_E = 16
_K = 2
_NC = 2
_NS = 16
_NW = _NC * _NS
_BM = 1024


def _logits_body(x_ref, w_ref, o_ref):
    o_ref[...] = lax.dot_general(
        x_ref[...], w_ref[...],
        (((1,), (1,)), ((), ())),
        preferred_element_type=jnp.float32,
    )


def _logits(x, W):
    T, N = x.shape
    return pl.pallas_call(
        _logits_body,
        grid=(T // _BM,),
        in_specs=[
            pl.BlockSpec((_BM, N), lambda i: (i, 0)),
            pl.BlockSpec((_E, N), lambda i: (0, 0)),
        ],
        out_specs=pl.BlockSpec((_BM, _E), lambda i: (i, 0)),
        out_shape=jax.ShapeDtypeStruct((T, _E), jnp.float32),
    )(x, W)


def _route_body(rw, logits_hbm, w_hbm, i_hbm, lg_v, w_v, i_v):
    wid = lax.axis_index("s") * _NC + lax.axis_index("c")
    base = wid * rw
    pltpu.sync_copy(logits_hbm.at[pl.ds(base, rw)], lg_v)
    iota = lax.iota(jnp.int32, 16)
    zeros = jnp.zeros((16,), jnp.float32)

    @plsc.parallel_loop(0, rw // 16, unroll=2)
    def tile(t):
        row = t * 16 + iota
        ls = [
            plsc.load_gather(lg_v, [row, jnp.full((16,), e, jnp.int32)])
            for e in range(_E)
        ]
        m = ls[0]
        for e in range(1, _E):
            m = jnp.maximum(m, ls[e])
        es = [jnp.exp(l - m) for l in ls]
        s = es[0]
        for e in range(1, _E):
            s = s + es[e]
        inv = 1.0 / s
        m1 = es[0] * inv
        i1 = jnp.zeros((16,), jnp.int32)
        m2 = jnp.full((16,), -1.0, jnp.float32)
        i2 = jnp.zeros((16,), jnp.int32)
        for e in range(1, _E):
            p = es[e] * inv
            gt1 = p > m1
            gt2 = p > m2
            i2 = jnp.where(gt1, i1, jnp.where(gt2, e, i2))
            m2 = jnp.where(gt1, m1, jnp.where(gt2, p, m2))
            i1 = jnp.where(gt1, e, i1)
            m1 = jnp.where(gt1, p, m1)
        for j in range(16):
            w_v[t * 16 + j, :] = zeros
        plsc.store_scatter(w_v, [row, i1], m1)
        plsc.store_scatter(w_v, [row, i2], m2)
        plsc.store_scatter(i_v, [row * _K], i1)
        plsc.store_scatter(i_v, [row * _K + 1], i2)

    pltpu.sync_copy(w_v, w_hbm.at[pl.ds(base, rw)])
    pltpu.sync_copy(i_v, i_hbm.at[pl.ds(base * _K, rw * _K)])


def _route(logits):
    T = logits.shape[0]
    rw = T // _NW
    mesh = plsc.VectorSubcoreMesh(core_axis_name="c", subcore_axis_name="s")
    weights, idx_flat = pl.kernel(
        functools.partial(_route_body, rw),
        out_type=[
            jax.ShapeDtypeStruct((T, _E), jnp.float32),
            jax.ShapeDtypeStruct((T * _K,), jnp.int32),
        ],
        mesh=mesh,
        compiler_params=pltpu.CompilerParams(
            needs_layout_passes=False, skip_device_barrier=True
        ),
        scratch_types=[
            pltpu.VMEM((rw, _E), jnp.float32),
            pltpu.VMEM((rw, _E), jnp.float32),
            pltpu.VMEM((rw * _K,), jnp.int32),
        ],
    )(logits)
    return weights, idx_flat.reshape(T, _K)


def kernel(x, W):
    logits = _logits(x, W)
    return _route(logits)
